# baseline (speedup 1.0000x reference)
import functools
import jax
import jax.numpy as jnp
from jax import lax
from jax.experimental import pallas as pl
from jax.experimental.pallas import tpu as pltpu
from jax.experimental.pallas import tpu_sc as plsc

EMB = 64
SCALE = 8.0
NC = 2
NS = 16
NW = NC * NS
G = 128
NBUF = 4
D = 2


@functools.partial(jax.jit, static_argnums=(2,))
def _transpose(table_t, tail2, v):
    hb = v // G
    ring = (hb // NW) * NW
    per_w = ring // NW
    extras = hb - ring
    mesh = plsc.VectorSubcoreMesh(core_axis_name="c", subcore_axis_name="s")

    @functools.partial(
        pl.kernel,
        mesh=mesh,
        out_type=jax.ShapeDtypeStruct((v // 2, 2 * EMB), jnp.float32),
        scratch_types=[
            pltpu.VMEM((2, EMB, G), jnp.float32),
            pltpu.VMEM((2, EMB, G), jnp.float32),
            pltpu.VMEM((EMB // 2, G), jnp.float32),
        ]
        + [pltpu.SemaphoreType.DMA] * 4,
        compiler_params=pltpu.CompilerParams(needs_layout_passes=False),
    )
    def k1(tab_hbm, tail_hbm, scr_hbm, src_v, dst_v, tl_v, *sems):
        in_sems = sems[:2]
        out_sems = sems[2:]
        wid = lax.axis_index("s") * NC + lax.axis_index("c")
        base = wid * per_w

        def get(i, b):
            return pltpu.async_copy(
                tab_hbm.at[:, pl.ds((base + i) * G, G)], src_v.at[b], in_sems[b]
            )

        def wait_get(b):
            pltpu.make_async_copy(
                tab_hbm.at[:, pl.ds(0, G)], src_v.at[b], in_sems[b]
            ).wait()

        def put(i, b):
            return pltpu.async_copy(
                dst_v.at[b], scr_hbm.at[pl.ds((base + i) * EMB, EMB)], out_sems[b]
            )

        def wait_put(b):
            pltpu.make_async_copy(
                dst_v.at[b], scr_hbm.at[pl.ds(0, EMB)], out_sems[b]
            ).wait()

        lanes = lax.iota(jnp.int32, 16)
        lanes9 = lanes * 9
        jv = [jc * 16 + lanes for jc in range(8)]
        pv = [(jc * 16 + lanes) >> 1 for jc in range(8)]
        qoff = [((jc * 16 + lanes) & 1) << 6 for jc in range(8)]

        def xpose(sb, db):
            def body(u, carry):
                for d in range(2):
                    cvec = (u * 2 + d + lanes9) & 63
                    vs = [
                        plsc.load_gather(src_v.at[sb], [cvec, jv[jc]])
                        for jc in range(8)
                    ]
                    for jc in range(8):
                        plsc.store_scatter(
                            dst_v.at[db], [pv[jc], cvec + qoff[jc]], vs[jc]
                        )
                return carry

            lax.fori_loop(0, EMB // 2, body, 0)

        get(0, 0)
        for i in range(2):
            if i + 1 < per_w:
                get(i + 1, 1 - i)
            wait_get(i)
            xpose(i, i)
            put(i, i)

        def steady(t, c):
            for b in range(2):
                i = 2 * t + b
                get(i + 1, 1 - b)
                wait_get(b)
                wait_put(b)
                xpose(b, b)
                put(i, b)
            return c

        lax.fori_loop(1, per_w // 2 - 1, steady, 0)

        for i in range(per_w - 2, per_w):
            b = i % 2
            if i + 1 < per_w:
                get(i + 1, 1 - b)
            wait_get(b)
            wait_put(b)
            xpose(b, b)
            put(i, b)
        for b in range(2):
            wait_put(b)

        for w in range(extras):

            @pl.when(wid == w)
            def _():
                h = ring + w
                pltpu.sync_copy(tab_hbm.at[:, pl.ds(h * G, G)], src_v.at[0])
                xpose(0, 0)
                pltpu.sync_copy(dst_v.at[0], scr_hbm.at[pl.ds(h * EMB, EMB)])

        @pl.when(wid == NW - 1)
        def _():
            pltpu.sync_copy(tail_hbm, tl_v)
            pltpu.sync_copy(tl_v, scr_hbm.at[pl.ds(hb * EMB, EMB // 2)])

    return k1(table_t, tail2)


@functools.partial(jax.jit, static_argnums=(2,))
def _run(idx3, table, ns):
    mesh = plsc.VectorSubcoreMesh(core_axis_name="c", subcore_axis_name="s")

    @functools.partial(
        pl.kernel,
        mesh=mesh,
        out_type=jax.ShapeDtypeStruct((ns, 8, NW, 8, G), jnp.float32),
        scratch_types=[
            pltpu.VMEM((ns, G), jnp.int32),
            pltpu.VMEM((NBUF, G, EMB), jnp.float32),
            pltpu.VMEM((NBUF, 8, 8, G), jnp.float32),
        ]
        + [pltpu.SemaphoreType.DMA] * (2 * NBUF),
        compiler_params=pltpu.CompilerParams(
            use_tc_tiling_on_sc=False, needs_layout_passes=False
        ),
    )
    def k(idx_hbm, table_hbm, out_hbm, idx_v, rows_v, stg_v, *sems):
        in_sems = sems[:NBUF]
        out_sems = sems[NBUF:]
        wid = lax.axis_index("s") * NC + lax.axis_index("c")
        pltpu.sync_copy(idx_hbm.at[wid], idx_v)

        def gather(s, b):
            return pltpu.async_copy(
                table_hbm.at[idx_v.at[s]], rows_v.at[b], in_sems[b]
            )

        def wait_gather(s, b):
            pltpu.make_async_copy(
                table_hbm.at[idx_v.at[s]], rows_v.at[b], in_sems[b]
            ).wait()

        def put(s, b):
            return pltpu.async_copy(
                stg_v.at[b], out_hbm.at[s, :, wid], out_sems[b]
            )

        def wait_put(b):
            pltpu.make_async_copy(
                stg_v.at[b], out_hbm.at[0, :, wid], out_sems[b]
            ).wait()

        lanes = lax.iota(jnp.int32, 16)
        lanes9 = lanes * 9
        rowv = [lc * 16 + lanes for lc in range(8)]

        def xpose_scale(b):
            def body(u, carry):
                for d in range(2):
                    cvec = (u * 2 + d + lanes9) & 63
                    gvec = cvec >> 3
                    clovec = cvec & 7
                    vs = [
                        plsc.load_gather(rows_v.at[b], [rowv[lc], cvec])
                        for lc in range(8)
                    ]
                    for lc in range(8):
                        plsc.store_scatter(
                            stg_v.at[b], [gvec, clovec, rowv[lc]], vs[lc] * SCALE
                        )
                return carry

            lax.fori_loop(0, EMB // 2, body, 0)

        gather(0, 0)
        gather(1, 1)
        for s in range(D):
            gather(s + D, s + D)
            wait_gather(s, s)
            xpose_scale(s)
            put(s, s)

        def steady(t, c):
            for b in range(NBUF):
                s = D + t * NBUF + b
                cb = (D + b) % NBUF
                wait_put(b)
                gather(s + D, b)
                wait_gather(s, cb)
                xpose_scale(cb)
                put(s, cb)
            return c

        lax.fori_loop(0, (ns - 2 * D) // NBUF, steady, 0)

        for i in range(D):
            s = ns - D + i
            cb = s % NBUF
            wait_gather(s, cb)
            xpose_scale(cb)
            put(s, cb)
        for b in range(NBUF):
            wait_put(b)

    return k(idx3, table)


def kernel(tokens, table):
    nb, ns = tokens.shape
    v = table.shape[0]
    idx3 = tokens.astype(jnp.int32).T.reshape(ns, NW, G).transpose(1, 0, 2)
    table_t = table.T
    tail_lo = (v // G) * G
    tail2 = table[tail_lo:].reshape((v - tail_lo) // 2, 2 * EMB)
    pairs = _transpose(table_t, tail2, v)
    table_lin = pairs.reshape(v, EMB)
    out5 = _run(idx3, table_lin, ns)
    return jnp.transpose(out5, (2, 4, 0, 1, 3)).reshape(nb, ns, EMB)


# speedup vs baseline: 1.1420x; 1.1420x over previous
import functools
import jax
import jax.numpy as jnp
from jax import lax
from jax.experimental import pallas as pl
from jax.experimental.pallas import tpu as pltpu
from jax.experimental.pallas import tpu_sc as plsc

EMB = 64
SCALE = 8.0
NC = 2
NS = 16
NW = NC * NS
G = 128
NBUF = 4
D = 2


@functools.partial(jax.jit, static_argnums=(2,))
def _transpose(table_t, tail2, v):
    hb = v // G
    ring = (hb // NW) * NW
    per_w = ring // NW
    extras = hb - ring
    mesh = plsc.VectorSubcoreMesh(core_axis_name="c", subcore_axis_name="s")

    @functools.partial(
        pl.kernel,
        mesh=mesh,
        out_type=jax.ShapeDtypeStruct((v // 2, 2 * EMB), jnp.float32),
        scratch_types=[
            pltpu.VMEM((4, EMB, G), jnp.float32),
            pltpu.VMEM((4, EMB, G), jnp.float32),
            pltpu.VMEM((EMB // 2, G), jnp.float32),
        ]
        + [pltpu.SemaphoreType.DMA] * 8,
        compiler_params=pltpu.CompilerParams(needs_layout_passes=False),
    )
    def k1(tab_hbm, tail_hbm, scr_hbm, src_v, dst_v, tl_v, *sems):
        in_sems = sems[:4]
        out_sems = sems[4:]
        wid = lax.axis_index("s") * NC + lax.axis_index("c")
        base = wid * per_w

        def get(i, b):
            return pltpu.async_copy(
                tab_hbm.at[:, pl.ds((base + i) * G, G)], src_v.at[b], in_sems[b]
            )

        def wait_get(b):
            pltpu.make_async_copy(
                tab_hbm.at[:, pl.ds(0, G)], src_v.at[b], in_sems[b]
            ).wait()

        def put(i, b):
            return pltpu.async_copy(
                dst_v.at[b], scr_hbm.at[pl.ds((base + i) * EMB, EMB)], out_sems[b]
            )

        def wait_put(b):
            pltpu.make_async_copy(
                dst_v.at[b], scr_hbm.at[pl.ds(0, EMB)], out_sems[b]
            ).wait()

        lanes = lax.iota(jnp.int32, 16)
        lanes9 = lanes * 9
        jv = [jc * 16 + lanes for jc in range(8)]
        pv = [(jc * 16 + lanes) >> 1 for jc in range(8)]
        qoff = [((jc * 16 + lanes) & 1) << 6 for jc in range(8)]

        def xpose(sb, db):
            def body(c0, carry):
                cvec = (c0 + lanes9) & 63
                vs = [
                    plsc.load_gather(src_v.at[sb], [cvec, jv[jc]])
                    for jc in range(8)
                ]
                for jc in range(8):
                    plsc.store_scatter(
                        dst_v.at[db], [pv[jc], cvec + qoff[jc]], vs[jc]
                    )
                return carry

            lax.fori_loop(0, EMB, body, 0)

        get(0, 0)
        get(1, 1)
        for i in range(2):
            get(i + 2, i + 2)
            wait_get(i)
            xpose(i, i)
            put(i, i)

        def steady(t, c):
            for b in range(4):
                i = 2 + 4 * t + b
                cb = (2 + b) % 4
                wait_put(b)
                get(i + 2, b)
                wait_get(cb)
                xpose(cb, cb)
                put(i, cb)
            return c

        lax.fori_loop(0, (per_w - 4) // 4, steady, 0)

        for i in range(per_w - 2, per_w):
            cb = i % 4
            wait_get(cb)
            xpose(cb, cb)
            put(i, cb)
        for b in range(4):
            wait_put(b)

        for w in range(extras):

            @pl.when(wid == w)
            def _():
                h = ring + w
                pltpu.sync_copy(tab_hbm.at[:, pl.ds(h * G, G)], src_v.at[0])
                xpose(0, 0)
                pltpu.sync_copy(dst_v.at[0], scr_hbm.at[pl.ds(h * EMB, EMB)])

        @pl.when(wid == NW - 1)
        def _():
            pltpu.sync_copy(tail_hbm, tl_v)
            pltpu.sync_copy(tl_v, scr_hbm.at[pl.ds(hb * EMB, EMB // 2)])

    return k1(table_t, tail2)


@functools.partial(jax.jit, static_argnums=(2,))
def _run(idx3, table, ns):
    mesh = plsc.VectorSubcoreMesh(core_axis_name="c", subcore_axis_name="s")

    @functools.partial(
        pl.kernel,
        mesh=mesh,
        out_type=jax.ShapeDtypeStruct((ns, 8, NW, 8, G), jnp.float32),
        scratch_types=[
            pltpu.VMEM((ns, G), jnp.int32),
            pltpu.VMEM((NBUF, G, EMB), jnp.float32),
            pltpu.VMEM((NBUF, 8, 8, G), jnp.float32),
        ]
        + [pltpu.SemaphoreType.DMA] * (2 * NBUF),
        compiler_params=pltpu.CompilerParams(
            use_tc_tiling_on_sc=False, needs_layout_passes=False
        ),
    )
    def k(idx_hbm, table_hbm, out_hbm, idx_v, rows_v, stg_v, *sems):
        in_sems = sems[:NBUF]
        out_sems = sems[NBUF:]
        wid = lax.axis_index("s") * NC + lax.axis_index("c")
        pltpu.sync_copy(idx_hbm.at[wid], idx_v)

        def gather(s, b):
            return pltpu.async_copy(
                table_hbm.at[idx_v.at[s]], rows_v.at[b], in_sems[b]
            )

        def wait_gather(s, b):
            pltpu.make_async_copy(
                table_hbm.at[idx_v.at[s]], rows_v.at[b], in_sems[b]
            ).wait()

        def put(s, b):
            return pltpu.async_copy(
                stg_v.at[b], out_hbm.at[s, :, wid], out_sems[b]
            )

        def wait_put(b):
            pltpu.make_async_copy(
                stg_v.at[b], out_hbm.at[0, :, wid], out_sems[b]
            ).wait()

        lanes = lax.iota(jnp.int32, 16)
        lanes9 = lanes * 9
        rowv = [lc * 16 + lanes for lc in range(8)]

        def xpose_scale(b):
            def body(c0, carry):
                cvec = (c0 + lanes9) & 63
                gvec = cvec >> 3
                clovec = cvec & 7
                vs = [
                    plsc.load_gather(rows_v.at[b], [rowv[lc], cvec])
                    for lc in range(8)
                ]
                for lc in range(8):
                    plsc.store_scatter(
                        stg_v.at[b], [gvec, clovec, rowv[lc]], vs[lc] * SCALE
                    )
                return carry

            lax.fori_loop(0, EMB, body, 0)

        gather(0, 0)
        gather(1, 1)
        for s in range(D):
            gather(s + D, s + D)
            wait_gather(s, s)
            xpose_scale(s)
            put(s, s)

        def steady(t, c):
            for b in range(NBUF):
                s = D + t * NBUF + b
                cb = (D + b) % NBUF
                wait_put(b)
                gather(s + D, b)
                wait_gather(s, cb)
                xpose_scale(cb)
                put(s, cb)
            return c

        lax.fori_loop(0, (ns - 2 * D) // NBUF, steady, 0)

        for i in range(D):
            s = ns - D + i
            cb = s % NBUF
            wait_gather(s, cb)
            xpose_scale(cb)
            put(s, cb)
        for b in range(NBUF):
            wait_put(b)

    return k(idx3, table)


def kernel(tokens, table):
    nb, ns = tokens.shape
    v = table.shape[0]
    idx3 = tokens.astype(jnp.int32).T.reshape(ns, NW, G).transpose(1, 0, 2)
    table_t = table.T
    tail_lo = (v // G) * G
    tail2 = table[tail_lo:].reshape((v - tail_lo) // 2, 2 * EMB)
    pairs = _transpose(table_t, tail2, v)
    table_lin = pairs.reshape(v, EMB)
    out5 = _run(idx3, table_lin, ns)
    return jnp.transpose(out5, (2, 4, 0, 1, 3)).reshape(nb, ns, EMB)
